# Initial kernel scaffold; baseline (speedup 1.0000x reference)
import jax
import jax.numpy as jnp
from jax.experimental import pallas as pl


def kernel(x0, Qt_bar, t):
    raise NotImplementedError("write your pallas kernel here")


# speedup vs baseline: 12.3527x; 12.3527x over previous
import functools

import jax
import jax.numpy as jnp
from jax import lax
from jax.experimental import pallas as pl
from jax.experimental.pallas import tpu as pltpu
from jax.experimental.pallas import tpu_sc as plsc

N_TOKENS = 819200
K = 8
T_STEPS = 300

NC = 2
NS = 16
NW = NC * NS
L = 16

TOK_PER_W = N_TOKENS // NW
BLK = 2560
NBLK = TOK_PER_W // BLK
GROUPS = BLK // L


def _sc_body(x_hbm, q_hbm, t_hbm, out_hbm, q_v, x_v, t_v, o_v):
    wid = lax.axis_index("s") * NC + lax.axis_index("c")

    pltpu.sync_copy(q_hbm, q_v)

    iota = lax.iota(jnp.int32, L)
    iota8 = iota * K

    def group(g, _):
        tt = t_v[pl.ds(g * L, L)]
        qbase = tt * (K * K)
        de = plsc.load_gather(q_v, [qbase])
        cc = plsc.load_gather(q_v, [qbase + K])
        dd = plsc.load_gather(q_v, [qbase + K + 1])
        xb = g * (L * K) + iota8
        x0c = plsc.load_gather(x_v, [xb])
        s = plsc.load_gather(x_v, [xb + 1])
        plsc.store_scatter(o_v, [xb + 1], s * dd)
        for j in range(2, K):
            xj = plsc.load_gather(x_v, [xb + j])
            s = s + xj
            plsc.store_scatter(o_v, [xb + j], xj * dd)
        plsc.store_scatter(o_v, [xb], x0c * de + cc * s)
        return 0

    def block(b, _):
        base = wid * TOK_PER_W + b * BLK
        pltpu.sync_copy(x_hbm.at[pl.ds(base * K, BLK * K)], x_v)
        pltpu.sync_copy(t_hbm.at[pl.ds(base, BLK)], t_v)
        lax.fori_loop(0, GROUPS, group, 0)
        pltpu.sync_copy(o_v, out_hbm.at[pl.ds(base * K, BLK * K)])
        return 0

    lax.fori_loop(0, NBLK, block, 0)


_sc_call = functools.partial(
    pl.kernel,
    mesh=plsc.VectorSubcoreMesh(core_axis_name="c", subcore_axis_name="s"),
    out_type=jax.ShapeDtypeStruct((N_TOKENS * K,), jnp.float32),
    scratch_types=[
        pltpu.VMEM((T_STEPS * K * K,), jnp.float32),
        pltpu.VMEM((BLK * K,), jnp.float32),
        pltpu.VMEM((BLK,), jnp.int32),
        pltpu.VMEM((BLK * K,), jnp.float32),
    ],
    compiler_params=pltpu.CompilerParams(needs_layout_passes=False),
)(_sc_body)


def kernel(x0, Qt_bar, t):
    out = _sc_call(x0.reshape(-1), Qt_bar.reshape(-1), t)
    return out.reshape(N_TOKENS, K)
